# baseline (speedup 1.0000x reference)
import functools

import jax
import jax.numpy as jnp
import numpy as np
from jax import lax
from jax.experimental import pallas as pl
from jax.experimental.pallas import tpu as pltpu
from jax.experimental.pallas import tpu_sc as plsc

_BAG = 64
_CLS = 128
_N = 16384
_NC = 2
_NS = 16
_NW = _NC * _NS
_ROWS_PER_W = _N // _NW
_CHUNK = 128
_NCHUNK = _ROWS_PER_W // _CHUNK


def _seg_body(yt_ref, yp_ref,
              sums_ref, cnts_ref,
              idx_v, rows_v, zb_v, ones_v, zc_v, acc_sh, cnt_sh, sem_rows, sem_sc):
    c = lax.axis_index("c")
    s = lax.axis_index("s")
    wid = s * _NC + c

    loads = []
    for g in range(_NCHUNK):
        loads.append(pltpu.async_copy(
            yp_ref.at[pl.ds(wid * _ROWS_PER_W + g * _CHUNK, _CHUNK)],
            rows_v.at[pl.ds(g * _CHUNK, _CHUNK)],
            sem_rows.at[g]))
    for g in range(_NCHUNK):
        pltpu.sync_copy(
            yt_ref.at[pl.ds(wid * _ROWS_PER_W + g * _CHUNK, _CHUNK)],
            idx_v.at[g])

    zeros16 = jnp.zeros((16,), jnp.float32)
    ones16 = jnp.ones((16,), jnp.float32)

    def _fill_zero_row(i, carry):
        for j in range(_CLS // 16):
            zb_v[i, pl.ds(j * 16, 16)] = zeros16
        return carry

    lax.fori_loop(0, _BAG, _fill_zero_row, 0)

    def _fill_ones_row(i, carry):
        ones_v[i, pl.ds(0, 16)] = ones16
        return carry

    lax.fori_loop(0, _CHUNK, _fill_ones_row, 0)

    def _fill_zc_row(i, carry):
        zc_v[i, pl.ds(0, 16)] = zeros16
        return carry

    lax.fori_loop(0, _BAG, _fill_zc_row, 0)

    pltpu.sync_copy(zb_v, acc_sh.at[s])
    pltpu.sync_copy(zc_v, cnt_sh.at[s])

    cps = []
    for g in range(_NCHUNK):
        loads[g].wait()
        cps.append(pltpu.async_copy(
            rows_v.at[pl.ds(g * _CHUNK, _CHUNK)],
            acc_sh.at[s].at[idx_v.at[g]], sem_sc, add=True))
        cps.append(pltpu.async_copy(
            ones_v, cnt_sh.at[s].at[idx_v.at[g]], sem_sc, add=True))
    for cp in cps:
        cp.wait()

    pltpu.sync_copy(acc_sh.at[s], sums_ref.at[wid])
    pltpu.sync_copy(cnt_sh.at[s], cnts_ref.at[wid])


_seg_kernel = functools.partial(
    pl.kernel,
    mesh=plsc.VectorSubcoreMesh(core_axis_name="c", subcore_axis_name="s"),
    out_type=[
        jax.ShapeDtypeStruct((_NW, _BAG, _CLS), jnp.float32),
        jax.ShapeDtypeStruct((_NW, _BAG, 16), jnp.float32),
    ],
    scratch_types=[
        pltpu.VMEM((_NCHUNK, _CHUNK), jnp.int32),
        pltpu.VMEM((_ROWS_PER_W, _CLS), jnp.float32),
        pltpu.VMEM((_BAG, _CLS), jnp.float32),
        pltpu.VMEM((_CHUNK, 16), jnp.float32),
        pltpu.VMEM((_BAG, 16), jnp.float32),
        pltpu.VMEM_SHARED((_NS, _BAG, _CLS), jnp.float32),
        pltpu.VMEM_SHARED((_NS, _BAG, 16), jnp.float32),
        pltpu.SemaphoreType.DMA((_NCHUNK,)),
        pltpu.SemaphoreType.DMA,
    ],
)(_seg_body)


def _finish_body(sums_ref, cnts_ref, theta_ref, out_ref):
    sums = jnp.sum(sums_ref[...], axis=0)
    cnts = jnp.sum(cnts_ref[...], axis=0)
    cnt = cnts[:, 0:1]
    means = sums / cnt
    m = jnp.max(means, axis=-1, keepdims=True)
    e = jnp.exp(means - m)
    se = jnp.sum(e, axis=-1, keepdims=True)
    sm = e / se
    theta_c = jnp.clip(theta_ref[...], 1e-7, 1.0 - 1e-7)
    loss = -theta_c * jnp.log(sm + 1e-7)
    out_ref[...] = jnp.sum(loss).reshape(1, 1)


def kernel(y_true, y_pred, theta):
    sums2, cnts2 = _seg_kernel(y_true.astype(jnp.int32), y_pred)
    out = pl.pallas_call(
        _finish_body,
        out_shape=jax.ShapeDtypeStruct((1, 1), jnp.float32),
    )(sums2, cnts2, theta.reshape(_BAG, 1))
    return out[0, 0]


# speedup vs baseline: 1.0486x; 1.0486x over previous
import functools

import jax
import jax.numpy as jnp
import numpy as np
from jax import lax
from jax.experimental import pallas as pl
from jax.experimental.pallas import tpu as pltpu
from jax.experimental.pallas import tpu_sc as plsc

_BAG = 64
_CLS = 128
_N = 16384
_NC = 2
_NS = 16
_NW = _NC * _NS
_ROWS_PER_W = _N // _NW
_CHUNK = 128
_NCHUNK = _ROWS_PER_W // _CHUNK


def _seg_body(yt_ref, yp_ref,
              sums_ref, cnts_ref,
              idx_v, rows_v, zb_v, ones_v, zc_v, acc_sh, cnt_sh, sem_rows, sem_sc,
              sem_init):
    c = lax.axis_index("c")
    s = lax.axis_index("s")
    wid = s * _NC + c

    loads = []
    for g in range(_NCHUNK):
        loads.append(pltpu.async_copy(
            yp_ref.at[pl.ds(wid * _ROWS_PER_W + g * _CHUNK, _CHUNK)],
            rows_v.at[pl.ds(g * _CHUNK, _CHUNK)],
            sem_rows.at[g]))
    pltpu.sync_copy(yt_ref.at[wid], idx_v)

    zeros16 = jnp.zeros((16,), jnp.float32)
    ones16 = jnp.ones((16,), jnp.float32)

    def _fill_zero_row(i, carry):
        for j in range(_CLS // 16):
            zb_v[i, pl.ds(j * 16, 16)] = zeros16
        return carry

    lax.fori_loop(0, _BAG, _fill_zero_row, 0)

    def _fill_ones_row(i, carry):
        ones_v[i, pl.ds(0, 16)] = ones16
        return carry

    lax.fori_loop(0, _CHUNK, _fill_ones_row, 0)

    def _fill_zc_row(i, carry):
        zc_v[i, pl.ds(0, 16)] = zeros16
        return carry

    lax.fori_loop(0, _BAG, _fill_zc_row, 0)

    cp_zb = pltpu.async_copy(zb_v, acc_sh.at[s], sem_init)
    cp_zc = pltpu.async_copy(zc_v, cnt_sh.at[s], sem_init)
    cp_zb.wait()
    cp_zc.wait()

    cps = []
    for g in range(_NCHUNK):
        loads[g].wait()
        cps.append(pltpu.async_copy(
            rows_v.at[pl.ds(g * _CHUNK, _CHUNK)],
            acc_sh.at[s].at[idx_v.at[g]], sem_sc, add=True))
        cps.append(pltpu.async_copy(
            ones_v, cnt_sh.at[s].at[idx_v.at[g]], sem_sc, add=True))
    for cp in cps:
        cp.wait()

    pltpu.sync_copy(acc_sh.at[s], sums_ref.at[wid])
    pltpu.sync_copy(cnt_sh.at[s], cnts_ref.at[wid])


_seg_kernel = functools.partial(
    pl.kernel,
    mesh=plsc.VectorSubcoreMesh(core_axis_name="c", subcore_axis_name="s"),
    out_type=[
        jax.ShapeDtypeStruct((_NW, _BAG, _CLS), jnp.float32),
        jax.ShapeDtypeStruct((_NW, _BAG, 16), jnp.float32),
    ],
    scratch_types=[
        pltpu.VMEM((_NCHUNK, _CHUNK), jnp.int32),
        pltpu.VMEM((_ROWS_PER_W, _CLS), jnp.float32),
        pltpu.VMEM((_BAG, _CLS), jnp.float32),
        pltpu.VMEM((_CHUNK, 16), jnp.float32),
        pltpu.VMEM((_BAG, 16), jnp.float32),
        pltpu.VMEM_SHARED((_NS, _BAG, _CLS), jnp.float32),
        pltpu.VMEM_SHARED((_NS, _BAG, 16), jnp.float32),
        pltpu.SemaphoreType.DMA((_NCHUNK,)),
        pltpu.SemaphoreType.DMA,
        pltpu.SemaphoreType.DMA,
    ],
)(_seg_body)


def _finish_body(sums_ref, cnts_ref, theta_ref, out_ref):
    sums = jnp.sum(sums_ref[...], axis=0)
    cnts = jnp.sum(cnts_ref[...], axis=0)
    cnt = cnts[:, 0:1]
    means = sums / cnt
    m = jnp.max(means, axis=-1, keepdims=True)
    e = jnp.exp(means - m)
    se = jnp.sum(e, axis=-1, keepdims=True)
    sm = e / se
    theta_c = jnp.clip(theta_ref[...], 1e-7, 1.0 - 1e-7)
    loss = -theta_c * jnp.log(sm + 1e-7)
    out_ref[...] = jnp.sum(loss).reshape(1, 1)


def kernel(y_true, y_pred, theta):
    yt2 = y_true.astype(jnp.int32).reshape(_NW, _NCHUNK, _CHUNK)
    sums2, cnts2 = _seg_kernel(yt2, y_pred)
    out = pl.pallas_call(
        _finish_body,
        out_shape=jax.ShapeDtypeStruct((1, 1), jnp.float32),
    )(sums2, cnts2, theta.reshape(_BAG, 1))
    return out[0, 0]
